# baseline (speedup 1.0000x reference)
import jax
import jax.numpy as jnp
from jax.experimental import pallas as pl
from jax.experimental.pallas import tpu as pltpu

_BM = 200


def _body(p_ref, x_ref, w_ref, o_ref):
    p = p_ref[...].astype(jnp.bfloat16)
    xb = x_ref[...].astype(jnp.bfloat16)
    s = jnp.dot(p, xb, preferred_element_type=jnp.float32)
    o = jnp.dot(s, w_ref[...], preferred_element_type=jnp.float32)
    o_ref[...] = jnp.maximum(o, 0.0)


def kernel(x, p_matrix, W):
    n, in_ch = x.shape
    out_ch = W.shape[1]
    grid = (pl.cdiv(n, _BM),)
    return pl.pallas_call(
        _body,
        grid=grid,
        in_specs=[
            pl.BlockSpec((_BM, n), lambda i: (i, 0)),
            pl.BlockSpec((n, in_ch), lambda i: (0, 0)),
            pl.BlockSpec((in_ch, out_ch), lambda i: (0, 0)),
        ],
        out_specs=pl.BlockSpec((_BM, out_ch), lambda i: (i, 0)),
        out_shape=jax.ShapeDtypeStruct((n, out_ch), jnp.float32),
        compiler_params=pltpu.CompilerParams(
            dimension_semantics=("arbitrary",),
        ),
    )(p_matrix, x, W)


# speedup vs baseline: 1.0338x; 1.0338x over previous
import jax
import jax.numpy as jnp
from jax.experimental import pallas as pl
from jax.experimental.pallas import tpu as pltpu

_BM = 256


def _body(p_ref, x_ref, w_ref, o_ref):
    p = p_ref[...].astype(jnp.bfloat16)
    xb = x_ref[...].astype(jnp.bfloat16)
    s = jnp.dot(p, xb, preferred_element_type=jnp.float32)
    o = jnp.dot(s, w_ref[...], preferred_element_type=jnp.float32)
    o_ref[...] = jnp.maximum(o, 0.0)


def kernel(x, p_matrix, W):
    n, in_ch = x.shape
    out_ch = W.shape[1]
    grid = (pl.cdiv(n, _BM),)
    return pl.pallas_call(
        _body,
        grid=grid,
        in_specs=[
            pl.BlockSpec((_BM, n), lambda i: (i, 0)),
            pl.BlockSpec((n, in_ch), lambda i: (0, 0)),
            pl.BlockSpec((in_ch, out_ch), lambda i: (0, 0)),
        ],
        out_specs=pl.BlockSpec((_BM, out_ch), lambda i: (i, 0)),
        out_shape=jax.ShapeDtypeStruct((n, out_ch), jnp.float32),
        compiler_params=pltpu.CompilerParams(
            dimension_semantics=("arbitrary",),
        ),
    )(p_matrix, x, W)
